# Initial kernel scaffold; baseline (speedup 1.0000x reference)
import jax
import jax.numpy as jnp
from jax.experimental import pallas as pl


def kernel(query_length, key_length, rel_embedding):
    raise NotImplementedError("write your pallas kernel here")


# speedup vs baseline: 41.9768x; 41.9768x over previous
import functools

import jax
import jax.numpy as jnp
from jax import lax
from jax.experimental import pallas as pl
from jax.experimental.pallas import tpu as pltpu
from jax.experimental.pallas import tpu_sc as plsc

N_HEADS = 16
N_BUCKETS = 32
MAX_DIST = 128
Q_LEN = 2048
K_LEN = 2048

NSHIFT = 8
TBL = 4096
LAG = 8


def _table_body(emb_ref, log16_ref, doff_ref, out_ref):
    r = lax.broadcasted_iota(jnp.int32, (NSHIFT, TBL), 0)
    c = lax.broadcasted_iota(jnp.int32, (NSHIFT, TBL), 1)
    relative_position = (r + c) - (Q_LEN - 1) + doff_ref[0, 0]
    n = -relative_position
    half = N_BUCKETS // 2
    big = jnp.where(n < 0, half, 0)
    n = jnp.abs(n)
    max_exact = half // 2
    nf = n.astype(jnp.float32)
    val_large = max_exact + (
        jnp.log(nf / max_exact) / log16_ref[0, 0] * (half - max_exact)
    ).astype(jnp.int32)
    val_large = jnp.minimum(val_large, half - 1)
    bucket = big + jnp.where(n < max_exact, n, val_large)
    for h in range(N_HEADS):
        acc = jnp.zeros((NSHIFT, TBL), jnp.float32)
        for b in range(N_BUCKETS):
            acc = acc + jnp.where(bucket == b, emb_ref[h, b], 0.0)
        out_ref[h] = acc


def _make_table(rel_embedding, log16, d_off):
    return pl.pallas_call(
        _table_body,
        out_shape=jax.ShapeDtypeStruct((N_HEADS, NSHIFT, TBL), jnp.float32),
        in_specs=[
            pl.BlockSpec(memory_space=pltpu.SMEM),
            pl.BlockSpec(memory_space=pltpu.SMEM),
            pl.BlockSpec(memory_space=pltpu.SMEM),
        ],
    )(rel_embedding, log16, d_off)


def _expand_body(tbl_hbm, out_hbm, sh_vmem, sem):
    h = lax.axis_index("s")
    half = lax.axis_index("c")
    rows = Q_LEN // 2
    base = half * rows

    pltpu.sync_copy(tbl_hbm.at[h], sh_vmem)

    def body(t, carry):
        row = base + t
        src = (Q_LEN - 1) - row
        rbit = src & (NSHIFT - 1)
        off = rbit * TBL + (src - rbit)
        off = pl.multiple_of(off, NSHIFT)
        pltpu.make_async_copy(
            sh_vmem.at[pl.ds(off, K_LEN)],
            out_hbm.at[h, row],
            sem,
        ).start()

        @pl.when(t >= LAG)
        def _():
            pltpu.make_async_copy(
                sh_vmem.at[pl.ds(0, K_LEN)],
                out_hbm.at[h, base],
                sem,
            ).wait()

        return carry

    lax.fori_loop(0, rows, body, 0)
    for _ in range(LAG):
        pltpu.make_async_copy(
            sh_vmem.at[pl.ds(0, K_LEN)],
            out_hbm.at[h, base],
            sem,
        ).wait()


@functools.cache
def _make_expand():
    return pl.kernel(
        _expand_body,
        out_type=jax.ShapeDtypeStruct((N_HEADS, Q_LEN, K_LEN), jnp.float32),
        mesh=plsc.VectorSubcoreMesh(core_axis_name="c", subcore_axis_name="s"),
        scratch_types=[
            pltpu.VMEM((NSHIFT * TBL,), jnp.float32),
            pltpu.SemaphoreType.DMA,
        ],
        compiler_params=pltpu.CompilerParams(use_tc_tiling_on_sc=False),
    )


def kernel(query_length, key_length, rel_embedding):
    d_off = jnp.asarray(
        (key_length - K_LEN) - (query_length - Q_LEN), jnp.int32
    ).reshape(1, 1)
    log16 = jnp.log(jnp.full((1, 1), MAX_DIST / (N_BUCKETS // 4), jnp.float32))
    tbl = _make_table(rel_embedding, log16, d_off)
    tbl = tbl.reshape(N_HEADS, NSHIFT * TBL)
    return _make_expand()(tbl)
